# baseline (speedup 1.0000x reference)
import functools

import jax
import jax.numpy as jnp
from jax import lax
from jax.experimental import pallas as pl
from jax.experimental.pallas import tpu as pltpu
from jax.experimental.pallas import tpu_sc as plsc

_D = 128
_P = 32
_NC = 2
_NS = 16
_NW = _NC * _NS
_BT = 128
_L = 16


def _emb_combine(idxT, posT, tab):
    s_len, b_total = idxT.shape
    n_btiles = b_total // _BT
    n_sgroups = _NW // n_btiles
    s_per_w = s_len // n_sgroups
    mesh = plsc.VectorSubcoreMesh(core_axis_name="c", subcore_axis_name="s")

    @functools.partial(
        pl.kernel,
        out_type=jax.ShapeDtypeStruct((s_len, _D + _P, b_total), jnp.float32),
        mesh=mesh,
        scratch_types=[
            pltpu.VMEM((s_per_w, _BT), jnp.int32),
            pltpu.VMEM((_BT, _D), jnp.float32),
            pltpu.VMEM((_BT, _D), jnp.float32),
            pltpu.VMEM((_D, _BT + 1), jnp.float32),
            pltpu.VMEM((_D, _BT + 1), jnp.float32),
            pltpu.VMEM((_P, _BT), jnp.float32),
            pltpu.VMEM((_P, _BT), jnp.float32),
            pltpu.SemaphoreType.DMA,
            pltpu.SemaphoreType.DMA,
            pltpu.SemaphoreType.DMA,
            pltpu.SemaphoreType.DMA,
            pltpu.SemaphoreType.DMA,
            pltpu.SemaphoreType.DMA,
            pltpu.SemaphoreType.DMA,
        ],
        compiler_params=pltpu.CompilerParams(needs_layout_passes=False),
    )
    def body(idx_hbm, pos_hbm, tab_hbm, out_hbm, idx_all,
             word_v0, word_v1, tr_v0, tr_v1, pos_v0, pos_v1,
             sg0, sg1, sp0, sp1, sw0, sw1, si):
        word_v = (word_v0, word_v1)
        tr_v = (tr_v0, tr_v1)
        pos_v = (pos_v0, pos_v1)
        sg = (sg0, sg1)
        sp = (sp0, sp1)
        sw = (sw0, sw1)

        wid = lax.axis_index("s") * _NC + lax.axis_index("c")
        b0 = pl.multiple_of((wid % n_btiles) * _BT, _BT)
        s0 = (wid // n_btiles) * s_per_w

        def start_inputs(i, slot):
            pltpu.async_copy(tab_hbm.at[idx_all.at[i]], word_v[slot], sg[slot])
            pltpu.async_copy(
                pos_hbm.at[s0 + i, pl.ds(0, _P), pl.ds(b0, _BT)],
                pos_v[slot], sp[slot])

        def wait_inputs(slot):
            pltpu.make_async_copy(
                tab_hbm.at[idx_all.at[0]], word_v[slot], sg[slot]).wait()
            pltpu.make_async_copy(
                pos_hbm.at[0, pl.ds(0, _P), pl.ds(0, _BT)],
                pos_v[slot], sp[slot]).wait()

        def start_writes(i, slot):
            pltpu.async_copy(
                pos_v[slot],
                out_hbm.at[s0 + i, pl.ds(_D, _P), pl.ds(b0, _BT)], sw[slot])
            pltpu.async_copy(
                tr_v[slot].at[pl.ds(0, _D), pl.ds(0, _BT)],
                out_hbm.at[s0 + i, pl.ds(0, _D), pl.ds(b0, _BT)], sw[slot])

        def wait_writes(slot):
            pltpu.make_async_copy(
                pos_v[slot],
                out_hbm.at[0, pl.ds(_D, _P), pl.ds(0, _BT)], sw[slot]).wait()
            pltpu.make_async_copy(
                tr_v[slot].at[pl.ds(0, _D), pl.ds(0, _BT)],
                out_hbm.at[0, pl.ds(0, _D), pl.ds(0, _BT)], sw[slot]).wait()

        fidx = [lax.iota(jnp.int32, _L) + fg * _L for fg in range(_D // _L)]

        def transpose(slot):
            wv, tv = word_v[slot], tr_v[slot]

            @plsc.parallel_loop(0, _BT, unroll=8,
                                carry=jnp.zeros((_L,), jnp.int32))
            def _bbody(b, bv):
                for fg in range(_D // _L):
                    plsc.store_scatter(
                        tv, [fidx[fg], bv], wv[b, pl.ds(fg * _L, _L)])
                return bv + 1

        def step(i, slot, first=False, last=False):
            if not first:
                wait_writes(1 - slot)
            if not last:
                start_inputs(i + 1, 1 - slot)
            wait_inputs(slot)
            transpose(slot)
            start_writes(i, slot)

        def stage(i, carry):
            pltpu.async_copy(
                idx_hbm.at[s0 + i, pl.ds(b0, _BT)], idx_all.at[i], si)
            return carry
        lax.fori_loop(0, s_per_w, stage, 0)

        def drain(i, carry):
            pltpu.make_async_copy(
                idx_hbm.at[0, pl.ds(0, _BT)], idx_all.at[0], si).wait()
            return carry
        lax.fori_loop(0, s_per_w, drain, 0)

        start_inputs(0, 0)
        step(0, 0, first=True)
        step(1, 1)

        def pair(j, carry):
            step(2 * j, 0)
            step(2 * j + 1, 1)
            return carry

        lax.fori_loop(1, s_per_w // 2 - 1, pair, 0)

        step(s_per_w - 2, 0)
        step(s_per_w - 1, 1, last=True)
        wait_writes(1)

    return body(idxT, posT, tab)


def kernel(token_ids, pos_onehot, W):
    idxT = token_ids.T.astype(jnp.int32)
    posT = jnp.transpose(pos_onehot, (1, 2, 0)).astype(jnp.float32)
    outT = _emb_combine(idxT, posT, W)
    return jnp.transpose(outT, (2, 0, 1))


# speedup vs baseline: 1.6683x; 1.6683x over previous
import functools

import jax
import jax.numpy as jnp
from jax import lax
from jax.experimental import pallas as pl
from jax.experimental.pallas import tpu as pltpu
from jax.experimental.pallas import tpu_sc as plsc

_D = 128
_P = 32
_NC = 2
_NS = 16
_NW = _NC * _NS
_BT = 128


def _emb_gather(idxT, tab):
    s_len, b_total = idxT.shape
    n_btiles = b_total // _BT
    n_sgroups = _NW // n_btiles
    s_per_w = s_len // n_sgroups
    mesh = plsc.VectorSubcoreMesh(core_axis_name="c", subcore_axis_name="s")

    @functools.partial(
        pl.kernel,
        out_type=jax.ShapeDtypeStruct((s_len, n_btiles, _BT, _D), jnp.float32),
        mesh=mesh,
        scratch_types=[
            pltpu.VMEM((s_per_w, _BT), jnp.int32),
            pltpu.VMEM((_BT, _D), jnp.float32),
            pltpu.VMEM((_BT, _D), jnp.float32),
            pltpu.SemaphoreType.DMA,
            pltpu.SemaphoreType.DMA,
            pltpu.SemaphoreType.DMA,
            pltpu.SemaphoreType.DMA,
            pltpu.SemaphoreType.DMA,
        ],
    )
    def body(idx_hbm, tab_hbm, out_hbm, idx_all,
             word_v0, word_v1, sg0, sg1, sw0, sw1, si):
        word_v = (word_v0, word_v1)
        sg = (sg0, sg1)
        sw = (sw0, sw1)

        wid = lax.axis_index("s") * _NC + lax.axis_index("c")
        bt = wid % n_btiles
        b0 = pl.multiple_of(bt * _BT, _BT)
        s0 = (wid // n_btiles) * s_per_w

        def start_inputs(i, slot):
            pltpu.async_copy(tab_hbm.at[idx_all.at[i]], word_v[slot], sg[slot])

        def wait_inputs(slot):
            pltpu.make_async_copy(
                tab_hbm.at[idx_all.at[0]], word_v[slot], sg[slot]).wait()

        def start_writes(i, slot):
            pltpu.async_copy(word_v[slot], out_hbm.at[s0 + i, bt], sw[slot])

        def wait_writes(slot):
            pltpu.make_async_copy(
                word_v[slot], out_hbm.at[0, 0], sw[slot]).wait()

        def step(i, slot, first=False, last=False):
            if not first:
                wait_writes(1 - slot)
            if not last:
                start_inputs(i + 1, 1 - slot)
            wait_inputs(slot)
            start_writes(i, slot)

        def stage(i, carry):
            pltpu.async_copy(
                idx_hbm.at[s0 + i, pl.ds(b0, _BT)], idx_all.at[i], si)
            return carry
        lax.fori_loop(0, s_per_w, stage, 0)

        def drain(i, carry):
            pltpu.make_async_copy(
                idx_hbm.at[0, pl.ds(0, _BT)], idx_all.at[0], si).wait()
            return carry
        lax.fori_loop(0, s_per_w, drain, 0)

        start_inputs(0, 0)
        step(0, 0, first=True)
        step(1, 1)

        def pair(j, carry):
            step(2 * j, 0)
            step(2 * j + 1, 1)
            return carry

        lax.fori_loop(1, s_per_w // 2 - 1, pair, 0)

        step(s_per_w - 2, 0)
        step(s_per_w - 1, 1, last=True)
        wait_writes(1)

    return body(idxT, tab)


def kernel(token_ids, pos_onehot, W):
    b, s = token_ids.shape
    idxT = token_ids.T.astype(jnp.int32)
    w4 = _emb_gather(idxT, W)
    word = jnp.transpose(w4, (1, 2, 0, 3)).reshape(b, s, _D)
    return jnp.concatenate([word, pos_onehot.astype(jnp.float32)], axis=-1)
